# baseline (speedup 1.0000x reference)
import functools

import jax
import jax.numpy as jnp
import numpy as np
from jax import lax
from jax.experimental import pallas as pl
from jax.experimental.pallas import tpu as pltpu
from jax.experimental.pallas import tpu_sc as plsc

B = 128
V = 100000
NC = 2
NS = 16
NW = NC * NS
LANES = 16
RPG = 8

V_MAIN = 99968
HALF_OFF = 49920
HALF_LEN = 50048
CHUNK = 3072
N_FULL = HALF_LEN // CHUNK
TAIL = HALF_LEN - N_FULL * CHUNK
NCHK = N_FULL + 1
TCOLS = V - V_MAIN
TFLAT = B * TCOLS
THRESH = 4.5
INT_MAX = 2**31 - 1

_cache = {}


def _constants(shape, dtype):
    ck = (tuple(shape), jnp.dtype(dtype).name)
    if ck not in _cache:
        with jax.ensure_compile_time_eval():
            key = jax.random.key(42)
            u = jax.random.uniform(key, shape, dtype=dtype,
                                   minval=jnp.finfo(dtype).tiny, maxval=1.0)
            g = -jnp.log(-jnp.log(u))
        gn = np.asarray(g)
        buckets = {}
        qmax = 1
        for h in range(2):
            base = h * HALF_OFF
            for c in range(NCHK):
                c0 = base + c * CHUNK
                cw = CHUNK if c < N_FULL else TAIL
                for row in range(B):
                    seg = gn[row, c0:c0 + cw]
                    cols = np.nonzero(seg >= THRESH)[0]
                    buckets[(h, c, row)] = cols
                    qmax = max(qmax, len(cols))
        Q = ((qmax + LANES - 1) // LANES) * LANES
        tcol = np.zeros((NW, RPG, NCHK, Q), np.int32)
        tg = np.zeros((NW, RPG, NCHK, Q), np.float32)
        for cid in range(NC):
            for sid in range(NS):
                group = cid * (NS // 2) + sid // 2
                h = sid % 2
                w = cid * NS + sid
                base = h * HALF_OFF
                for rr in range(RPG):
                    row = group * RPG + rr
                    for c in range(NCHK):
                        cols = buckets[(h, c, row)]
                        col0 = base + c * CHUNK
                        if len(cols) == 0:
                            cols = np.zeros((1,), np.int64)
                        pad = np.full(Q, cols[0], np.int64)
                        pad[:len(cols)] = cols[:Q]
                        tcol[w, rr, c] = pad.astype(np.int32)
                        tg[w, rr, c] = gn[row, col0 + pad]
        _cache[ck] = (
            jax.new_ref(g),
            jax.new_ref(jnp.asarray(g[:, V_MAIN:].reshape(-1))),
            jax.new_ref(jnp.asarray(tcol.reshape(-1))),
            jax.new_ref(jnp.asarray(tg.reshape(-1))),
            Q,
        )
    return _cache[ck]


def _make_body(Q):
    slab = RPG * NCHK * Q

    def body(logits_hbm, gumbel_hbm, tail_l_hbm, tail_g_hbm,
             tcol_hbm, tg_hbm, out_hbm,
             lbuf0, lbuf1, gbuf0, gbuf1, gtl2, tbl, tbg, tcb, tgb,
             cvb, cib, outv, tmpa, tmpb, sha, shb, shc,
             sem0, sem1, sem2, sem3):
        cid = lax.axis_index("c")
        sid = lax.axis_index("s")
        group = cid * (NS // 2) + sid // 2
        half = sid % 2
        wid = cid * NS + sid
        row0 = pl.multiple_of(group * RPG, RPG)
        col0 = pl.multiple_of(half * HALF_OFF, 128)
        lane = lax.iota(jnp.int32, LANES)
        lbufs, sems = (lbuf0, lbuf1), (sem0, sem1)

        toff = pl.multiple_of(wid * slab, LANES)
        pre = (pltpu.async_copy(tcol_hbm.at[pl.ds(toff, slab)], tcb, sem2),
               pltpu.async_copy(tg_hbm.at[pl.ds(toff, slab)], tgb, sem2),
               pltpu.async_copy(tail_l_hbm, tbl, sem2),
               pltpu.async_copy(tail_g_hbm, tbg, sem2))

        def issue(c):
            cw = CHUNK if c < N_FULL else TAIL
            src = (pl.ds(row0, RPG), pl.ds(col0 + c * CHUNK, cw))
            return pltpu.async_copy(logits_hbm.at[src],
                                    lbufs[c % 2] if c < N_FULL else gbuf0,
                                    sems[c % 2])

        descs = {0: issue(0)}
        mx = [jnp.full((LANES,), -jnp.inf, jnp.float32) for _ in range(RPG)]
        for rr in range(RPG):
            cvb[pl.ds(rr * LANES, LANES)] = jnp.full((LANES,), -jnp.inf,
                                                     jnp.float32)
            cib[pl.ds(rr * LANES, LANES)] = jnp.full((LANES,), INT_MAX,
                                                     jnp.int32)
        for d in pre:
            d.wait()
        for c in range(NCHK):
            if c + 1 < NCHK:
                descs[c + 1] = issue(c + 1)
            descs.pop(c).wait()
            buf = lbufs[c % 2] if c < N_FULL else gbuf0
            cw = CHUNK if c < N_FULL else TAIL

            def mbody(i, carry, buf=buf):
                out = []
                base = i * LANES
                for rr in range(RPG):
                    out.append(jnp.maximum(carry[rr],
                                           buf[rr, pl.ds(base, LANES)]))
                return tuple(out)

            mx = list(lax.fori_loop(0, cw // LANES, mbody, tuple(mx)))

            ccol0 = col0 + c * CHUNK

            def crow(rr, _unused, buf=buf, c=c):
                cv = cvb[pl.ds(rr * LANES, LANES)]
                ci = cib[pl.ds(rr * LANES, LANES)]
                rvec = jnp.full((LANES,), 0, jnp.int32) + rr
                tb = (rr * NCHK + c) * Q

                def cq(q, carry):
                    cv, ci = carry
                    off = tb + q * LANES
                    colv = tcb[pl.ds(off, LANES)]
                    gv = tgb[pl.ds(off, LANES)]
                    lv = plsc.load_gather(buf, [rvec, colv])
                    v = lv + gv
                    iv = colv + ccol0
                    take = (v > cv) | ((v == cv) & (iv < ci))
                    return (jnp.where(take, v, cv),
                            jnp.where(take, iv, ci))

                cv, ci = lax.fori_loop(0, Q // LANES, cq, (cv, ci))
                cvb[pl.ds(rr * LANES, LANES)] = cv
                cib[pl.ds(rr * LANES, LANES)] = ci
                return 0

            lax.fori_loop(0, RPG, crow, 0)

        tbase = pl.multiple_of(group * (RPG * TCOLS), LANES)
        mlv = jnp.full((LANES,), -jnp.inf, jnp.float32)
        bval = jnp.full((LANES,), -jnp.inf, jnp.float32)
        bidx = jnp.full((LANES,), INT_MAX, jnp.int32)
        for rr in range(RPG):
            cv = cvb[pl.ds(rr * LANES, LANES)]
            ci = cib[pl.ds(rr * LANES, LANES)]
            for kk in range(TCOLS // LANES):
                off = tbase + rr * TCOLS + kk * LANES
                v = tbl[pl.ds(off, LANES)] + tbg[pl.ds(off, LANES)]
                iv = lane + (V_MAIN + kk * LANES)
                take = (v > cv) | ((v == cv) & (iv < ci))
                cv = jnp.where(take, v, cv)
                ci = jnp.where(take, iv, ci)
            mlv = jnp.where(lane == rr, jnp.max(mx[rr]), mlv)
            m = jnp.max(cv)
            bval = jnp.where(lane == rr, m, bval)
            bidx = jnp.where(lane == rr,
                             jnp.min(jnp.where(cv == m, ci, INT_MAX)), bidx)

        my = pl.multiple_of(sid * LANES, LANES)
        pr = pl.multiple_of((sid ^ 1) * LANES, LANES)
        tmpa[...] = bval
        pltpu.sync_copy(tmpa, sha.at[pl.ds(my, LANES)])
        tmpb[...] = bidx
        pltpu.sync_copy(tmpb, shb.at[pl.ds(my, LANES)])
        tmpa[...] = mlv
        pltpu.sync_copy(tmpa, shc.at[pl.ds(my, LANES)])
        plsc.subcore_barrier()
        pltpu.sync_copy(sha.at[pl.ds(pr, LANES)], tmpa)
        pv = tmpa[...]
        pltpu.sync_copy(shb.at[pl.ds(pr, LANES)], tmpb)
        pi = tmpb[...]
        pltpu.sync_copy(shc.at[pl.ds(pr, LANES)], tmpa)
        pml = tmpa[...]
        take = (pv > bval) | ((pv == bval) & (pi < bidx))
        bval = jnp.where(take, pv, bval)
        bidx = jnp.where(take, pi, bidx)

        notdone = (mlv + THRESH >= bval) | (pml + THRESH >= bval)
        any_nd = jnp.max(jnp.where(lane < RPG, notdone.astype(jnp.int32),
                                   0)) > 0

        @pl.when(any_nd)
        def _fallback():
            fb = [(jnp.full((LANES,), -jnp.inf, jnp.float32),
                   jnp.full((LANES,), INT_MAX, jnp.int32))
                  for _ in range(RPG)]
            flat = []
            for bv2, bi2 in fb:
                flat += [bv2, bi2]

            def fchunk(c, carry):
                cw = CHUNK
                csl = pl.multiple_of(c * CHUNK, 128)
                src = (pl.ds(row0, RPG), pl.ds(col0 + csl, cw))
                pltpu.sync_copy(logits_hbm.at[src], lbuf0)
                pltpu.sync_copy(gumbel_hbm.at[src], gbuf1)

                def fbody(i, carry2, c=c):
                    out = []
                    base = i * LANES
                    iv = lane + (col0 + c * CHUNK + base)
                    for rr in range(RPG):
                        bv2, bi2 = carry2[2 * rr], carry2[2 * rr + 1]
                        v = (lbuf0[rr, pl.ds(base, LANES)]
                             + gbuf1[rr, pl.ds(base, LANES)])
                        tk = (v > bv2) | ((v == bv2) & (iv < bi2))
                        out.append(jnp.where(tk, v, bv2))
                        out.append(jnp.where(tk, iv, bi2))
                    return tuple(out)

                return lax.fori_loop(0, cw // LANES, fbody, carry)

            flat = lax.fori_loop(0, N_FULL, fchunk, tuple(flat))
            src = (pl.ds(row0, RPG),
                   pl.ds(col0 + N_FULL * CHUNK, TAIL))
            pltpu.sync_copy(logits_hbm.at[src], gbuf0)
            pltpu.sync_copy(gumbel_hbm.at[src], gtl2)

            def tbody(i, carry2):
                out = []
                base = i * LANES
                iv = lane + (col0 + N_FULL * CHUNK + base)
                for rr in range(RPG):
                    bv2, bi2 = carry2[2 * rr], carry2[2 * rr + 1]
                    v = (gbuf0[rr, pl.ds(base, LANES)]
                         + gtl2[rr, pl.ds(base, LANES)])
                    tk = (v > bv2) | ((v == bv2) & (iv < bi2))
                    out.append(jnp.where(tk, v, bv2))
                    out.append(jnp.where(tk, iv, bi2))
                return tuple(out)

            flat = lax.fori_loop(0, TAIL // LANES, tbody, flat)
            fbv = jnp.full((LANES,), -jnp.inf, jnp.float32)
            fbi = jnp.full((LANES,), INT_MAX, jnp.int32)
            val = bval
            idx = bidx
            for rr in range(RPG):
                bv2, bi2 = flat[2 * rr], flat[2 * rr + 1]
                for kk in range(TCOLS // LANES):
                    off = tbase + rr * TCOLS + kk * LANES
                    v = tbl[pl.ds(off, LANES)] + tbg[pl.ds(off, LANES)]
                    iv = lane + (V_MAIN + kk * LANES)
                    tk = (v > bv2) | ((v == bv2) & (iv < bi2))
                    bv2 = jnp.where(tk, v, bv2)
                    bi2 = jnp.where(tk, iv, bi2)
                m = jnp.max(bv2)
                bi = jnp.min(jnp.where(bv2 == m, bi2, INT_MAX))
                fbv = jnp.where(lane == rr, m, fbv)
                fbi = jnp.where(lane == rr, bi, fbi)
            use = notdone
            tmpa[...] = jnp.where(use, fbv, val)
            tmpb[...] = jnp.where(use, fbi, idx)

        @pl.when(jnp.logical_not(any_nd))
        def _fast():
            tmpa[...] = bval
            tmpb[...] = bidx

        pltpu.sync_copy(tmpa, sha.at[pl.ds(my, LANES)])
        pltpu.sync_copy(tmpb, shb.at[pl.ds(my, LANES)])
        plsc.subcore_barrier()
        pltpu.sync_copy(sha.at[pl.ds(pr, LANES)], tmpa)
        bval = tmpa[...]
        pltpu.sync_copy(shb.at[pl.ds(pr, LANES)], tmpb)
        bidx = tmpb[...]
        pltpu.sync_copy(sha.at[pl.ds(my, LANES)], tmpa)
        mval = tmpa[...]
        pltpu.sync_copy(shb.at[pl.ds(my, LANES)], tmpb)
        midx = tmpb[...]
        take = (bval > mval) | ((bval == mval) & (bidx < midx))
        final = jnp.where(take, bidx, midx)

        @pl.when(half == 0)
        def _():
            outv[...] = final
            o_off = pl.multiple_of(group * LANES, LANES)
            pltpu.sync_copy(outv, out_hbm.at[pl.ds(o_off, LANES)])

    return body


@functools.cache
def _build_kernel(Q):
    slab = RPG * NCHK * Q
    return pl.kernel(
        _make_body(Q),
        out_type=jax.ShapeDtypeStruct((16 * LANES,), jnp.int32),
        mesh=plsc.VectorSubcoreMesh(core_axis_name="c", subcore_axis_name="s",
                                    num_cores=NC, num_subcores=NS),
        scratch_types=[
            pltpu.VMEM((RPG, CHUNK), jnp.float32),
            pltpu.VMEM((RPG, CHUNK), jnp.float32),
            pltpu.VMEM((RPG, TAIL), jnp.float32),
            pltpu.VMEM((RPG, CHUNK), jnp.float32),
            pltpu.VMEM((RPG, TAIL), jnp.float32),
            pltpu.VMEM((TFLAT,), jnp.float32),
            pltpu.VMEM((TFLAT,), jnp.float32),
            pltpu.VMEM((slab,), jnp.int32),
            pltpu.VMEM((slab,), jnp.float32),
            pltpu.VMEM((RPG * LANES,), jnp.float32),
            pltpu.VMEM((RPG * LANES,), jnp.int32),
            pltpu.VMEM((LANES,), jnp.int32),
            pltpu.VMEM((LANES,), jnp.float32),
            pltpu.VMEM((LANES,), jnp.int32),
            pltpu.VMEM_SHARED((NS * LANES,), jnp.float32),
            pltpu.VMEM_SHARED((NS * LANES,), jnp.int32),
            pltpu.VMEM_SHARED((NS * LANES,), jnp.float32),
            pltpu.SemaphoreType.DMA,
            pltpu.SemaphoreType.DMA,
            pltpu.SemaphoreType.DMA,
            pltpu.SemaphoreType.DMA,
        ],
        compiler_params=pltpu.CompilerParams(needs_layout_passes=False),
    )


def kernel(logits):
    assert logits.shape == (B, V)
    g, tail_g, tcol, tg, Q = _constants(logits.shape, logits.dtype)
    tail_l = logits[:, V_MAIN:].reshape(-1)
    out = _build_kernel(Q)(logits, g, tail_l, tail_g, tcol, tg)
    idx = out.reshape(16, LANES)[:, :RPG].reshape(B)
    return idx[:, None].astype(jnp.int64)


# speedup vs baseline: 1.4459x; 1.4459x over previous
import functools

import jax
import jax.numpy as jnp
import numpy as np
from jax import lax
from jax.experimental import pallas as pl
from jax.experimental.pallas import tpu as pltpu
from jax.experimental.pallas import tpu_sc as plsc

B = 128
V = 100000
NC = 2
NS = 16
NW = NC * NS
LANES = 16
RPG = 8

V_MAIN = 99968
HALF_OFF = 49920
HALF_LEN = 50048
CHUNK = 3072
N_FULL = HALF_LEN // CHUNK
TAIL = HALF_LEN - N_FULL * CHUNK
NCHK = N_FULL + 1
TCOLS = V - V_MAIN
TFLAT = B * TCOLS
THRESH = 4.5
INT_MAX = 2**31 - 1

_cache = {}


def _constants(shape, dtype):
    ck = (tuple(shape), jnp.dtype(dtype).name)
    if ck not in _cache:
        with jax.ensure_compile_time_eval():
            key = jax.random.key(42)
            u = jax.random.uniform(key, shape, dtype=dtype,
                                   minval=jnp.finfo(dtype).tiny, maxval=1.0)
            g = -jnp.log(-jnp.log(u))
        gn = np.asarray(g)
        buckets = {}
        qmax = 1
        for h in range(2):
            base = h * HALF_OFF
            for c in range(NCHK):
                c0 = base + c * CHUNK
                cw = CHUNK if c < N_FULL else TAIL
                for row in range(B):
                    cols = np.nonzero(gn[row, c0:c0 + cw] >= THRESH)[0]
                    buckets[(h, c, row)] = cols
                    qmax = max(qmax, len(cols))
        Q = ((qmax + LANES - 1) // LANES) * LANES
        tcol = np.zeros((NW, RPG, NCHK, Q), np.int32)
        tg = np.zeros((NW, RPG, NCHK, Q), np.float32)
        for cid in range(NC):
            for sid in range(NS):
                group = cid * (NS // 2) + sid // 2
                h = sid % 2
                w = cid * NS + sid
                base = h * HALF_OFF
                for rr in range(RPG):
                    row = group * RPG + rr
                    for c in range(NCHK):
                        cols = buckets[(h, c, row)]
                        col0 = base + c * CHUNK
                        if len(cols) == 0:
                            cols = np.zeros((1,), np.int64)
                        pad = np.full(Q, cols[0], np.int64)
                        pad[:len(cols)] = cols[:Q]
                        tcol[w, rr, c] = pad.astype(np.int32)
                        tg[w, rr, c] = gn[row, col0 + pad]
        _cache[ck] = (
            g,
            jnp.asarray(g[:, V_MAIN:].reshape(-1)),
            jnp.asarray(tcol.reshape(-1)),
            jnp.asarray(tg.reshape(-1)),
            Q,
        )
    return _cache[ck]


def _make_fast_body(Q):
    slab = RPG * NCHK * Q

    def body(logits_hbm, tail_l_hbm, tail_g_hbm, tcol_hbm, tg_hbm,
             out_hbm, nd_hbm,
             lbuf0, lbuf1, tbuf, tbl, tbg, tcb, tgb, cvb, cib,
             outv, ndv, tmpa, tmpb, sha, shb, shc, sem0, sem1, sem2):
        cid = lax.axis_index("c")
        sid = lax.axis_index("s")
        group = cid * (NS // 2) + sid // 2
        half = sid % 2
        wid = cid * NS + sid
        row0 = pl.multiple_of(group * RPG, RPG)
        col0 = pl.multiple_of(half * HALF_OFF, 128)
        lane = lax.iota(jnp.int32, LANES)
        lbufs, sems = (lbuf0, lbuf1), (sem0, sem1)

        toff = pl.multiple_of(wid * slab, LANES)
        pre = (pltpu.async_copy(tcol_hbm.at[pl.ds(toff, slab)], tcb, sem2),
               pltpu.async_copy(tg_hbm.at[pl.ds(toff, slab)], tgb, sem2),
               pltpu.async_copy(tail_l_hbm, tbl, sem2),
               pltpu.async_copy(tail_g_hbm, tbg, sem2))

        def issue(c):
            cw = CHUNK if c < N_FULL else TAIL
            src = (pl.ds(row0, RPG), pl.ds(col0 + c * CHUNK, cw))
            return pltpu.async_copy(logits_hbm.at[src],
                                    lbufs[c % 2] if c < N_FULL else tbuf,
                                    sems[c % 2])

        descs = {0: issue(0)}
        mx = [jnp.full((LANES,), -jnp.inf, jnp.float32) for _ in range(RPG)]
        for rr in range(RPG):
            cvb[pl.ds(rr * LANES, LANES)] = jnp.full((LANES,), -jnp.inf,
                                                     jnp.float32)
            cib[pl.ds(rr * LANES, LANES)] = jnp.full((LANES,), INT_MAX,
                                                     jnp.int32)
        for d in pre:
            d.wait()
        for c in range(NCHK):
            if c + 1 < NCHK:
                descs[c + 1] = issue(c + 1)
            descs.pop(c).wait()
            buf = lbufs[c % 2] if c < N_FULL else tbuf
            cw = CHUNK if c < N_FULL else TAIL

            def mbody(i, carry, buf=buf):
                out = []
                base = i * LANES
                for rr in range(RPG):
                    out.append(jnp.maximum(carry[rr],
                                           buf[rr, pl.ds(base, LANES)]))
                return tuple(out)

            mx = list(lax.fori_loop(0, cw // LANES, mbody, tuple(mx)))

            ccol0 = col0 + c * CHUNK

            def crow(rr, _unused, buf=buf, c=c):
                cv = cvb[pl.ds(rr * LANES, LANES)]
                ci = cib[pl.ds(rr * LANES, LANES)]
                rvec = jnp.full((LANES,), 0, jnp.int32) + rr
                tb = (rr * NCHK + c) * Q

                def cq(q, carry):
                    cv, ci = carry
                    off = tb + q * LANES
                    colv = tcb[pl.ds(off, LANES)]
                    gv = tgb[pl.ds(off, LANES)]
                    lv = plsc.load_gather(buf, [rvec, colv])
                    v = lv + gv
                    iv = colv + ccol0
                    take = (v > cv) | ((v == cv) & (iv < ci))
                    return (jnp.where(take, v, cv),
                            jnp.where(take, iv, ci))

                cv, ci = lax.fori_loop(0, Q // LANES, cq, (cv, ci))
                cvb[pl.ds(rr * LANES, LANES)] = cv
                cib[pl.ds(rr * LANES, LANES)] = ci
                return 0

            lax.fori_loop(0, RPG, crow, 0)

        tbase = pl.multiple_of(group * (RPG * TCOLS), LANES)
        mlv = jnp.full((LANES,), -jnp.inf, jnp.float32)
        bval = jnp.full((LANES,), -jnp.inf, jnp.float32)
        bidx = jnp.full((LANES,), INT_MAX, jnp.int32)
        for rr in range(RPG):
            cv = cvb[pl.ds(rr * LANES, LANES)]
            ci = cib[pl.ds(rr * LANES, LANES)]
            for kk in range(TCOLS // LANES):
                off = tbase + rr * TCOLS + kk * LANES
                v = tbl[pl.ds(off, LANES)] + tbg[pl.ds(off, LANES)]
                iv = lane + (V_MAIN + kk * LANES)
                take = (v > cv) | ((v == cv) & (iv < ci))
                cv = jnp.where(take, v, cv)
                ci = jnp.where(take, iv, ci)
            mlv = jnp.where(lane == rr, jnp.max(mx[rr]), mlv)
            m = jnp.max(cv)
            bval = jnp.where(lane == rr, m, bval)
            bidx = jnp.where(lane == rr,
                             jnp.min(jnp.where(cv == m, ci, INT_MAX)), bidx)

        my = pl.multiple_of(sid * LANES, LANES)
        pr = pl.multiple_of((sid ^ 1) * LANES, LANES)
        tmpa[...] = bval
        pltpu.sync_copy(tmpa, sha.at[pl.ds(my, LANES)])
        tmpb[...] = bidx
        pltpu.sync_copy(tmpb, shb.at[pl.ds(my, LANES)])
        tmpa[...] = mlv
        pltpu.sync_copy(tmpa, shc.at[pl.ds(my, LANES)])
        plsc.subcore_barrier()
        pltpu.sync_copy(sha.at[pl.ds(pr, LANES)], tmpa)
        pv = tmpa[...]
        pltpu.sync_copy(shb.at[pl.ds(pr, LANES)], tmpb)
        pi = tmpb[...]
        pltpu.sync_copy(shc.at[pl.ds(pr, LANES)], tmpa)
        pml = tmpa[...]
        take = (pv > bval) | ((pv == bval) & (pi < bidx))
        bval = jnp.where(take, pv, bval)
        bidx = jnp.where(take, pi, bidx)

        notdone = (mlv + THRESH >= bval) | (pml + THRESH >= bval)
        nd32 = jnp.where(lane < RPG, notdone.astype(jnp.int32), 0)

        @pl.when(half == 0)
        def _():
            outv[...] = bidx
            ndv[...] = nd32
            o_off = pl.multiple_of(group * LANES, LANES)
            pltpu.sync_copy(outv, out_hbm.at[pl.ds(o_off, LANES)])
            pltpu.sync_copy(ndv, nd_hbm.at[pl.ds(o_off, LANES)])

    return body


def _full_scan_body(logits_hbm, gumbel_hbm, tail_l_hbm, tail_g_hbm, out_hbm,
                    lbuf0, lbuf1, gbuf0, gbuf1, ltail, gtail, tbl, tbg,
                    outv, tmpv, tmpi, shv, shi, sem0, sem1, sem2):
    cid = lax.axis_index("c")
    sid = lax.axis_index("s")
    group = cid * (NS // 2) + sid // 2
    half = sid % 2
    row0 = pl.multiple_of(group * RPG, RPG)
    col0 = pl.multiple_of(half * HALF_OFF, 128)
    lane = lax.iota(jnp.int32, LANES)
    lbufs, gbufs, sems = (lbuf0, lbuf1), (gbuf0, gbuf1), (sem0, sem1)

    tail_descs = (pltpu.async_copy(tail_l_hbm, tbl, sem2),
                  pltpu.async_copy(tail_g_hbm, tbg, sem2))

    def issue(c):
        slot = c % 2
        src = (pl.ds(row0, RPG), pl.ds(col0 + c * CHUNK, CHUNK))
        dl = pltpu.async_copy(logits_hbm.at[src], lbufs[slot], sems[slot])
        dg = pltpu.async_copy(gumbel_hbm.at[src], gbufs[slot], sems[slot])
        return (dl, dg)

    def scan_chunk(lb, gb, cols, col0_vec, states):
        def body(i, carry):
            out = []
            base = i * LANES
            iv = col0_vec + base
            for rr in range(RPG):
                bv, bi = carry[2 * rr], carry[2 * rr + 1]
                v = lb[rr, pl.ds(base, LANES)] + gb[rr, pl.ds(base, LANES)]
                take = (v > bv) | ((v == bv) & (iv < bi))
                out.append(jnp.where(take, v, bv))
                out.append(jnp.where(take, iv, bi))
            return tuple(out)

        flat = []
        for bv, bi in states:
            flat += [bv, bi]
        flat = lax.fori_loop(0, cols // LANES, body, tuple(flat))
        return [(flat[2 * rr], flat[2 * rr + 1]) for rr in range(RPG)]

    descs = {0: issue(0)}
    states = [(jnp.full((LANES,), -jnp.inf, jnp.float32),
               jnp.full((LANES,), INT_MAX, jnp.int32)) for _ in range(RPG)]
    for c in range(N_FULL):
        if c + 1 < N_FULL:
            descs[c + 1] = issue(c + 1)
        else:
            tsrc = (pl.ds(row0, RPG), pl.ds(col0 + N_FULL * CHUNK, TAIL))
            descs[N_FULL] = (
                pltpu.async_copy(logits_hbm.at[tsrc], ltail, sem0),
                pltpu.async_copy(gumbel_hbm.at[tsrc], gtail, sem0))
        for d in descs.pop(c):
            d.wait()
        slot = c % 2
        states = scan_chunk(lbufs[slot], gbufs[slot], CHUNK,
                            lane + (col0 + c * CHUNK), states)
    for d in descs.pop(N_FULL):
        d.wait()
    states = scan_chunk(ltail, gtail, TAIL,
                        lane + (col0 + N_FULL * CHUNK), states)

    for d in tail_descs:
        d.wait()
    tbase = pl.multiple_of(group * (RPG * TCOLS), LANES)
    for rr in range(RPG):
        bv, bi = states[rr]
        for kk in range(TCOLS // LANES):
            off = tbase + rr * TCOLS + kk * LANES
            v = tbl[pl.ds(off, LANES)] + tbg[pl.ds(off, LANES)]
            iv = lane + (V_MAIN + kk * LANES)
            take = v > bv
            bv = jnp.where(take, v, bv)
            bi = jnp.where(take, iv, bi)
        states[rr] = (bv, bi)

    val8 = jnp.full((LANES,), -jnp.inf, jnp.float32)
    idx8 = jnp.full((LANES,), INT_MAX, jnp.int32)
    for rr in range(RPG):
        bv, bi = states[rr]
        m = jnp.max(bv)
        best = jnp.min(jnp.where(bv == m, bi, INT_MAX))
        val8 = jnp.where(lane == rr, m, val8)
        idx8 = jnp.where(lane == rr, best, idx8)

    my = pl.multiple_of(sid * LANES, LANES)
    pr = pl.multiple_of((sid ^ 1) * LANES, LANES)
    tmpv[...] = val8
    pltpu.sync_copy(tmpv, shv.at[pl.ds(my, LANES)])
    tmpi[...] = idx8
    pltpu.sync_copy(tmpi, shi.at[pl.ds(my, LANES)])
    plsc.subcore_barrier()
    pltpu.sync_copy(shv.at[pl.ds(pr, LANES)], tmpv)
    pltpu.sync_copy(shi.at[pl.ds(pr, LANES)], tmpi)
    pv = tmpv[...]
    pi = tmpi[...]
    take = (pv > val8) | ((pv == val8) & (pi < idx8))
    idx8 = jnp.where(take, pi, idx8)

    @pl.when(half == 0)
    def _():
        outv[...] = idx8
        o_off = pl.multiple_of(group * LANES, LANES)
        pltpu.sync_copy(outv, out_hbm.at[pl.ds(o_off, LANES)])


_MESH = dict(core_axis_name="c", subcore_axis_name="s",
             num_cores=NC, num_subcores=NS)


@functools.cache
def _build_fast(Q):
    slab = RPG * NCHK * Q
    return pl.kernel(
        _make_fast_body(Q),
        out_type=(jax.ShapeDtypeStruct((16 * LANES,), jnp.int32),
                  jax.ShapeDtypeStruct((16 * LANES,), jnp.int32)),
        mesh=plsc.VectorSubcoreMesh(**_MESH),
        scratch_types=[
            pltpu.VMEM((RPG, CHUNK), jnp.float32),
            pltpu.VMEM((RPG, CHUNK), jnp.float32),
            pltpu.VMEM((RPG, TAIL), jnp.float32),
            pltpu.VMEM((TFLAT,), jnp.float32),
            pltpu.VMEM((TFLAT,), jnp.float32),
            pltpu.VMEM((slab,), jnp.int32),
            pltpu.VMEM((slab,), jnp.float32),
            pltpu.VMEM((RPG * LANES,), jnp.float32),
            pltpu.VMEM((RPG * LANES,), jnp.int32),
            pltpu.VMEM((LANES,), jnp.int32),
            pltpu.VMEM((LANES,), jnp.int32),
            pltpu.VMEM((LANES,), jnp.float32),
            pltpu.VMEM((LANES,), jnp.int32),
            pltpu.VMEM_SHARED((NS * LANES,), jnp.float32),
            pltpu.VMEM_SHARED((NS * LANES,), jnp.int32),
            pltpu.VMEM_SHARED((NS * LANES,), jnp.float32),
            pltpu.SemaphoreType.DMA,
            pltpu.SemaphoreType.DMA,
            pltpu.SemaphoreType.DMA,
        ],
        compiler_params=pltpu.CompilerParams(needs_layout_passes=False),
    )


@functools.cache
def _build_full():
    return pl.kernel(
        _full_scan_body,
        out_type=jax.ShapeDtypeStruct((16 * LANES,), jnp.int32),
        mesh=plsc.VectorSubcoreMesh(**_MESH),
        scratch_types=[
            pltpu.VMEM((RPG, CHUNK), jnp.float32),
            pltpu.VMEM((RPG, CHUNK), jnp.float32),
            pltpu.VMEM((RPG, CHUNK), jnp.float32),
            pltpu.VMEM((RPG, CHUNK), jnp.float32),
            pltpu.VMEM((RPG, TAIL), jnp.float32),
            pltpu.VMEM((RPG, TAIL), jnp.float32),
            pltpu.VMEM((TFLAT,), jnp.float32),
            pltpu.VMEM((TFLAT,), jnp.float32),
            pltpu.VMEM((LANES,), jnp.int32),
            pltpu.VMEM((LANES,), jnp.float32),
            pltpu.VMEM((LANES,), jnp.int32),
            pltpu.VMEM_SHARED((NS * LANES,), jnp.float32),
            pltpu.VMEM_SHARED((NS * LANES,), jnp.int32),
            pltpu.SemaphoreType.DMA,
            pltpu.SemaphoreType.DMA,
            pltpu.SemaphoreType.DMA,
        ],
        compiler_params=pltpu.CompilerParams(needs_layout_passes=False),
    )


def kernel(logits):
    assert logits.shape == (B, V)
    g, tail_g, tcol, tg, Q = _constants(logits.shape, logits.dtype)
    tail_l = logits[:, V_MAIN:].reshape(-1)
    idx, nd = _build_fast(Q)(logits, tail_l, tail_g, tcol, tg)
    any_nd = jnp.sum(nd) > 0
    idx = lax.cond(any_nd,
                   lambda: _build_full()(logits, g, tail_l, tail_g),
                   lambda: idx)
    out = idx.reshape(16, LANES)[:, :RPG].reshape(B)
    return out[:, None].astype(jnp.int64)
